# baseline (speedup 1.0000x reference)
import jax
import jax.numpy as jnp
from jax.experimental import pallas as pl
from jax.experimental.pallas import tpu as pltpu

_G, _T, _D = 8, 16, 128
_N = 10000
_E = 320000
_TT = _G * _T
_INNER_PRUNE = 0.01
_CROSS_PRUNE = 0.1
_B = 1024
_NBLK = (_T + _N + _B - 1) // _B
_EB = _E // _NBLK


def _dense_body(xc_ref, tok_ref, wt_ref,
                adj_ref, tx_ref, carry_ref):
    i = pl.program_id(0)
    xc = xc_ref[...]
    tok = tok_ref[...]
    wt = wt_ref[...]

    prev_tail = jnp.where(i == 0, tok, carry_ref[...])
    carry_ref[...] = xc[_B - _TT:]

    rows = jnp.concatenate([prev_tail[_TT - _T:], xc[:_B - _T]], axis=0)

    logits = jax.lax.dot_general(wt, rows, (((1,), (1,)), ((), ())),
                                 preferred_element_type=jnp.float32)
    best = jnp.full((1, _B), -jnp.inf, dtype=jnp.float32)
    route = jnp.zeros((1, _B), dtype=jnp.int32)
    for g in range(_G):
        lg = logits[g:g + 1, :]
        upd = lg > best
        best = jnp.where(upd, lg, best)
        route = jnp.where(upd, g, route)

    dots = jax.lax.dot_general(tok, rows, (((1,), (1,)), ((), ())),
                               preferred_element_type=jnp.float32)
    sim = jax.nn.sigmoid(dots)
    pruned = jnp.where(sim < _CROSS_PRUNE, 0.0, sim)
    gidx = jax.lax.broadcasted_iota(jnp.int32, (_TT, 1), 0) // _T
    adj_ref[...] = jnp.where(gidx == route, pruned, 0.0)

    @pl.when(i == 0)
    def _():
        gram = jax.lax.dot_general(tok, tok, (((1,), (1,)), ((), ())),
                                   preferred_element_type=jnp.float32)
        gsim = jax.nn.sigmoid(gram)
        gpruned = jnp.where(gsim < _INNER_PRUNE, 0.0, gsim)
        for g in range(_G):
            adj_ref[g * _T:(g + 1) * _T, 0:_T] = (
                gpruned[g * _T:(g + 1) * _T, g * _T:(g + 1) * _T])

    tx_ref[...] = jnp.concatenate([prev_tail, xc[:_B - _TT]], axis=0)


def _dense_call(x, tok, wt, interpret=False):
    return pl.pallas_call(
        _dense_body,
        grid=(_NBLK,),
        in_specs=[
            pl.BlockSpec((_B, _D), lambda i: (i, 0)),
            pl.BlockSpec((_TT, _D), lambda i: (0, 0)),
            pl.BlockSpec((_G, _D), lambda i: (0, 0)),
        ],
        out_specs=[
            pl.BlockSpec((_TT, _B), lambda i: (0, i)),
            pl.BlockSpec((_B, _D), lambda i: (i, 0)),
        ],
        out_shape=[
            jax.ShapeDtypeStruct((_TT, _T + _N), jnp.float32),
            jax.ShapeDtypeStruct((_TT + _N, _D), jnp.float32),
        ],
        scratch_shapes=[pltpu.VMEM((_TT, _D), jnp.float32)],
        interpret=interpret,
    )(x, tok, wt)


def kernel(x, tokens, pseudo_W, edge_index):
    tok = tokens.reshape(_TT, _D)
    wt = pseudo_W.T
    adj2d, total_x = _dense_call(x, tok, wt)
    adj = adj2d.reshape(_G, _T, _T + _N)
    return adj, total_x, edge_index + _TT


# speedup vs baseline: 1.5321x; 1.5321x over previous
import jax
import jax.numpy as jnp
from jax.experimental import pallas as pl
from jax.experimental.pallas import tpu as pltpu

_G, _T, _D = 8, 16, 128
_N = 10000
_E = 320000
_TT = _G * _T
_INNER_PRUNE = 0.01
_CROSS_PRUNE = 0.1
_B = 2048
_NBLK = (_T + _N + _B - 1) // _B
_EB = _E // _NBLK


def _dense_body(xc_ref, tok_ref, wt_ref, e_ref,
                adj_ref, tx_ref, ge_ref, carry_ref):
    i = pl.program_id(0)
    xc = xc_ref[...]
    tok = tok_ref[...]
    wt = wt_ref[...]

    ge_ref[...] = e_ref[...] + _TT

    prev_tail = jnp.where(i == 0, tok, carry_ref[...])
    carry_ref[...] = xc[_B - _TT:]

    rows = jnp.concatenate([prev_tail[_TT - _T:], xc[:_B - _T]], axis=0)

    logits = jax.lax.dot_general(wt, rows, (((1,), (1,)), ((), ())),
                                 preferred_element_type=jnp.float32)
    best = jnp.full((1, _B), -jnp.inf, dtype=jnp.float32)
    route = jnp.zeros((1, _B), dtype=jnp.int32)
    for g in range(_G):
        lg = logits[g:g + 1, :]
        upd = lg > best
        best = jnp.where(upd, lg, best)
        route = jnp.where(upd, g, route)

    dots = jax.lax.dot_general(tok, rows, (((1,), (1,)), ((), ())),
                               preferred_element_type=jnp.float32)
    sim = jax.nn.sigmoid(dots)
    pruned = jnp.where(sim < _CROSS_PRUNE, 0.0, sim)
    gidx = jax.lax.broadcasted_iota(jnp.int32, (_TT, 1), 0) // _T
    adj_ref[...] = jnp.where(gidx == route, pruned, 0.0)

    @pl.when(i == 0)
    def _():
        gram = jax.lax.dot_general(tok, tok, (((1,), (1,)), ((), ())),
                                   preferred_element_type=jnp.float32)
        gsim = jax.nn.sigmoid(gram)
        gpruned = jnp.where(gsim < _INNER_PRUNE, 0.0, gsim)
        for g in range(_G):
            adj_ref[g * _T:(g + 1) * _T, 0:_T] = (
                gpruned[g * _T:(g + 1) * _T, g * _T:(g + 1) * _T])

    tx_ref[...] = jnp.concatenate([prev_tail, xc[:_B - _TT]], axis=0)


def _dense_call(x, tok, wt, edge_index, interpret=False):
    return pl.pallas_call(
        _dense_body,
        grid=(_NBLK,),
        in_specs=[
            pl.BlockSpec((_B, _D), lambda i: (i, 0)),
            pl.BlockSpec((_TT, _D), lambda i: (0, 0)),
            pl.BlockSpec((_G, _D), lambda i: (0, 0)),
            pl.BlockSpec((2, _EB), lambda i: (0, i)),
        ],
        out_specs=[
            pl.BlockSpec((_TT, _B), lambda i: (0, i)),
            pl.BlockSpec((_B, _D), lambda i: (i, 0)),
            pl.BlockSpec((2, _EB), lambda i: (0, i)),
        ],
        out_shape=[
            jax.ShapeDtypeStruct((_TT, _T + _N), jnp.float32),
            jax.ShapeDtypeStruct((_TT + _N, _D), jnp.float32),
            jax.ShapeDtypeStruct((2, _E), edge_index.dtype),
        ],
        scratch_shapes=[pltpu.VMEM((_TT, _D), jnp.float32)],
        interpret=interpret,
    )(x, tok, wt, edge_index)


def kernel(x, tokens, pseudo_W, edge_index):
    tok = tokens.reshape(_TT, _D)
    wt = pseudo_W.T
    adj2d, total_x, g_edge_index = _dense_call(x, tok, wt, edge_index)
    adj = adj2d.reshape(_G, _T, _T + _N)
    return adj, total_x, g_edge_index
